# baseline (speedup 1.0000x reference)
import functools

import jax
import jax.numpy as jnp
from jax import lax
from jax.experimental import pallas as pl
from jax.experimental.pallas import tpu as pltpu
from jax.experimental.pallas import tpu_sc as plsc

_CHUNK = 128
_K = 8


@functools.partial(jax.jit, static_argnums=(2, 3))
def _sc_gather(table, tok2d, n_workers, iters):
    n_rows, _ = tok2d.shape
    t_total = n_rows * _CHUNK
    d = table.shape[1]
    per_w_rows = n_rows // n_workers
    step = _K * _CHUNK

    mesh = plsc.VectorSubcoreMesh(core_axis_name="c", subcore_axis_name="s")

    @functools.partial(
        pl.kernel,
        mesh=mesh,
        out_type=jax.ShapeDtypeStruct((t_total, d), jnp.float32),
        scratch_types=[
            pltpu.VMEM((_K, _CHUNK), jnp.int32),
            pltpu.VMEM((_K * _CHUNK, d), jnp.float32),
            pltpu.SemaphoreType.DMA,
        ],
        compiler_params=pltpu.CompilerParams(use_tc_tiling_on_sc=False),
    )
    def k(table_hbm, tok_hbm, out_hbm, idx_v, rows_v, sem):
        n_cores = lax.axis_size("c")
        wid = lax.axis_index("s") * n_cores + lax.axis_index("c")
        row_base = wid * per_w_rows

        def body(g, carry):
            r0 = row_base + g * _K
            pltpu.sync_copy(tok_hbm.at[pl.ds(r0, _K)], idx_v)
            copies = [
                pltpu.async_copy(
                    table_hbm.at[idx_v.at[j]],
                    rows_v.at[pl.ds(j * _CHUNK, _CHUNK)],
                    sem,
                )
                for j in range(_K)
            ]
            for c in copies:
                c.wait()
            pltpu.sync_copy(rows_v, out_hbm.at[pl.ds(r0 * _CHUNK, step)])
            return carry

        lax.fori_loop(0, iters, body, 0)

    return k(table, tok2d)


def _mm_body(g_ref, w_ref, b_ref, o_ref):
    acc = lax.dot_general(
        g_ref[...], w_ref[...], (((1,), (1,)), ((), ())),
        preferred_element_type=jnp.float32,
    )
    o_ref[...] = jnp.maximum(acc + b_ref[...], 0.0)


@jax.jit
def _tc_matmul(g, fc_w, fc_b2d):
    b, kdim = g.shape
    out_dim = fc_w.shape[0]
    bm = 128
    return pl.pallas_call(
        _mm_body,
        grid=(b // bm,),
        in_specs=[
            pl.BlockSpec((bm, kdim), lambda i: (i, 0)),
            pl.BlockSpec((out_dim, kdim), lambda i: (0, 0)),
            pl.BlockSpec((1, out_dim), lambda i: (0, 0)),
        ],
        out_specs=pl.BlockSpec((bm, out_dim), lambda i: (i, 0)),
        out_shape=jax.ShapeDtypeStruct((b, out_dim), jnp.float32),
    )(g, fc_w, fc_b2d)


def kernel(tokens, embed_table, fc_w, fc_b):
    batch, seq = tokens.shape
    d = embed_table.shape[1]
    t_total = batch * seq
    n_workers = 32
    iters = t_total // (n_workers * _K * _CHUNK)

    tok2d = tokens.reshape(t_total // _CHUNK, _CHUNK).astype(jnp.int32)
    gathered = _sc_gather(embed_table, tok2d, n_workers, iters)
    g = gathered.reshape(batch, seq * d)
    return _tc_matmul(g, fc_w, fc_b.reshape(1, d))
